# Initial kernel scaffold; baseline (speedup 1.0000x reference)
import jax
import jax.numpy as jnp
from jax.experimental import pallas as pl


def kernel(x, edge_index, W1, b1, Wmu, bmu, Wlv, blv):
    raise NotImplementedError("write your pallas kernel here")


# speedup vs baseline: 17.2248x; 17.2248x over previous
import functools

import jax
import jax.numpy as jnp
from jax import lax
from jax.experimental import pallas as pl
from jax.experimental.pallas import tpu as pltpu
from jax.experimental.pallas import tpu_sc as plsc

NC, NS, LANES = 2, 16, 16
NW = NC * NS


def _fill_rows(ref, rows, d, val):
    vec = jnp.full((LANES,), val, jnp.float32)

    def body(r, carry):
        for j in range(d // LANES):
            ref[r, pl.ds(j * LANES, LANES)] = vec
        return carry

    lax.fori_loop(0, rows, body, 0)


def _chunking(per_w):
    k = 80 if per_w % 80 == 0 else 8
    return k, per_w // k


def _make_sc_scatter(n, e, d):
    per_w = e // NW
    k, nchunk = _chunking(per_w)
    n_pad = ((n + 127) // 128) * 128
    rpt = n_pad // NS

    mesh = plsc.VectorSubcoreMesh(core_axis_name="c", subcore_axis_name="s")

    zr = 8

    scratch = [
        pltpu.VMEM((nchunk, k), jnp.int32),
        pltpu.VMEM((nchunk, k), jnp.int32),
        pltpu.VMEM((k, d), jnp.float32),
        pltpu.VMEM((zr, d), jnp.float32),
        pltpu.VMEM_SHARED((n_pad, d), jnp.float32),
        pltpu.SemaphoreType.DMA,
    ]

    def body(vals_hbm, src_hbm, dst_hbm, out_hbm, src_v, dst_v, rows, zbuf, acc, sem):
        c = lax.axis_index("c")
        s = lax.axis_index("s")
        wid = c * NS + s

        pltpu.sync_copy(dst_hbm.at[wid], dst_v)
        pltpu.sync_copy(src_hbm.at[wid], src_v)

        _fill_rows(zbuf, zr, d, 0.0)
        r0 = s * rpt

        def zero(i, carry):
            pltpu.sync_copy(zbuf, acc.at[pl.ds(r0 + i * zr, zr)])
            return carry

        lax.fori_loop(0, rpt // zr, zero, 0)
        plsc.subcore_barrier()

        def chunk(j, carry):
            pltpu.async_copy(vals_hbm.at[src_v.at[j]], rows, sem).wait()
            pltpu.sync_copy(rows, acc.at[dst_v.at[j]], add=True)
            return carry

        lax.fori_loop(0, nchunk, chunk, 0)
        plsc.subcore_barrier()

        pltpu.sync_copy(acc.at[pl.ds(r0, rpt)], out_hbm.at[c].at[pl.ds(r0, rpt)])

    return pl.kernel(
        body,
        out_type=jax.ShapeDtypeStruct((NC, n_pad, d), jnp.float32),
        mesh=mesh,
        scratch_types=scratch,
    )


def _make_sc_degree(n, e):
    per_w = e // NW
    k, nchunk = _chunking(per_w)
    n_pad = ((n + 2047) // 2048) * 2048
    rpt = n_pad // NS

    mesh = plsc.VectorSubcoreMesh(core_axis_name="c", subcore_axis_name="s")

    scratch = [
        pltpu.VMEM((nchunk, k), jnp.int32),
        pltpu.VMEM((k,), jnp.float32),
        pltpu.VMEM((rpt,), jnp.float32),
        pltpu.VMEM_SHARED((n_pad,), jnp.float32),
        pltpu.SemaphoreType.DMA,
    ]

    def body(dst_hbm, out_hbm, dst_v, ones_v, zbuf, acc, sem):
        c = lax.axis_index("c")
        s = lax.axis_index("s")
        wid = c * NS + s

        pltpu.sync_copy(dst_hbm.at[wid], dst_v)

        def fill1(r, carry):
            ones_v[pl.ds(r * LANES, LANES)] = jnp.ones((LANES,), jnp.float32)
            return carry
        lax.fori_loop(0, k // LANES, fill1, 0)

        def fill0(r, carry):
            zbuf[pl.ds(r * LANES, LANES)] = jnp.zeros((LANES,), jnp.float32)
            return carry
        lax.fori_loop(0, rpt // LANES, fill0, 0)

        r0 = s * rpt
        pltpu.sync_copy(zbuf, acc.at[pl.ds(r0, rpt)])
        plsc.subcore_barrier()

        def chunk(j, carry):
            pltpu.sync_copy(ones_v, acc.at[dst_v.at[j]], add=True)
            return carry

        lax.fori_loop(0, nchunk, chunk, 0)
        plsc.subcore_barrier()

        pltpu.sync_copy(acc.at[pl.ds(r0, rpt)], zbuf)
        pltpu.sync_copy(zbuf, out_hbm.at[c].at[pl.ds(r0, rpt)])

    return pl.kernel(
        body,
        out_type=jax.ShapeDtypeStruct((NC, n_pad), jnp.float32),
        mesh=mesh,
        scratch_types=scratch,
    )


def _rowscale(xblk, dis_t):
    g = dis_t.shape[1]
    parts = [xblk[128 * i:128 * (i + 1), :] * dis_t[:, i:i + 1] for i in range(g)]
    return jnp.concatenate(parts, axis=0)


def _tc_a_body(x_ref, w1_ref, degp_ref, hs1_ref, dist_ref):
    deg = 1.0 + degp_ref[0] + degp_ref[1]
    dis_t = jnp.transpose(1.0 / jnp.sqrt(deg))
    dist_ref[0] = dis_t
    xw = jnp.dot(x_ref[...], w1_ref[...], preferred_element_type=jnp.float32)
    hs1_ref[...] = _rowscale(xw, dis_t)


def _tc_b_body(a_ref, hs1_ref, dist_ref, b1_ref, wc_ref, o_ref, dh):
    dis_t = dist_ref[0]
    agg = a_ref[0] + a_ref[1] + hs1_ref[...]
    h = jnp.maximum(_rowscale(agg, dis_t) + b1_ref[...], 0.0)
    hw = jnp.dot(h, wc_ref[...], preferred_element_type=jnp.float32)
    hs2 = _rowscale(hw, dis_t)
    o_ref[...] = jnp.concatenate([hs2] * (dh // hs2.shape[1]), axis=1)


def _tc_c_body(a_ref, hs2_ref, dist_ref, bc_ref, mu_ref, lv_ref, dz):
    dis_t = dist_ref[0]
    agg = a_ref[0] + a_ref[1]
    mu_ref[...] = _rowscale(agg[:, :dz] + hs2_ref[:, :dz], dis_t) + bc_ref[:, :dz]
    lv_ref[...] = _rowscale(agg[:, dz:2 * dz] + hs2_ref[:, dz:2 * dz],
                            dis_t) + bc_ref[:, dz:]


def _tc_adj_body(zi_ref, zj_ref, o_ref):
    logits = lax.dot_general(
        zi_ref[...], zj_ref[...], (((1,), (1,)), ((), ())),
        preferred_element_type=jnp.float32)
    o_ref[...] = jax.nn.sigmoid(logits)


def kernel(x, edge_index, W1, b1, Wmu, bmu, Wlv, blv):
    n, din = x.shape
    e = edge_index.shape[1]
    dh = W1.shape[1]
    dz = Wmu.shape[1]
    d2 = 2 * dz

    k, nchunk = _chunking(e // NW)
    src_w = edge_index[0].reshape(NW, nchunk, k)
    dst_w = edge_index[1].reshape(NW, nchunk, k)

    wc = jnp.concatenate([Wmu, Wlv], axis=1)
    bc = jnp.concatenate([bmu, blv]).reshape(1, d2)
    b1r = b1.reshape(1, dh)

    n_pad_deg = ((n + 2047) // 2048) * 2048

    degp = _make_sc_degree(n, e)(dst_w)
    degp = degp.reshape(NC, n_pad_deg // 128, 128)

    bm = 1024
    g = bm // 128
    gi = pl.cdiv(n, bm)
    row = lambda i: (i, 0)
    halfrow = lambda i: (0, i, 0)

    hs1, dist = pl.pallas_call(
        _tc_a_body,
        grid=(gi,),
        in_specs=[
            pl.BlockSpec((bm, din), row),
            pl.BlockSpec((din, dh), lambda i: (0, 0)),
            pl.BlockSpec((NC, g, 128), halfrow),
        ],
        out_specs=[
            pl.BlockSpec((bm, dh), row),
            pl.BlockSpec((1, 128, g), lambda i: (i, 0, 0)),
        ],
        out_shape=[
            jax.ShapeDtypeStruct((n, dh), jnp.float32),
            jax.ShapeDtypeStruct((gi, 128, g), jnp.float32),
        ],
    )(x, W1, degp)

    acc1 = _make_sc_scatter(n, e, dh)(hs1, src_w, dst_w)

    hs2 = pl.pallas_call(
        functools.partial(_tc_b_body, dh=dh),
        grid=(gi,),
        in_specs=[
            pl.BlockSpec((NC, bm, dh), halfrow),
            pl.BlockSpec((bm, dh), row),
            pl.BlockSpec((1, 128, g), lambda i: (i, 0, 0)),
            pl.BlockSpec((1, dh), lambda i: (0, 0)),
            pl.BlockSpec((dh, d2), lambda i: (0, 0)),
        ],
        out_specs=pl.BlockSpec((bm, dh), row),
        out_shape=jax.ShapeDtypeStruct((n, dh), jnp.float32),
    )(acc1, hs1, dist, b1r, wc)

    acc2 = _make_sc_scatter(n, e, dh)(hs2, src_w, dst_w)

    mu, logvar = pl.pallas_call(
        functools.partial(_tc_c_body, dz=dz),
        grid=(gi,),
        in_specs=[
            pl.BlockSpec((NC, bm, dh), halfrow),
            pl.BlockSpec((bm, dh), row),
            pl.BlockSpec((1, 128, g), lambda i: (i, 0, 0)),
            pl.BlockSpec((1, d2), lambda i: (0, 0)),
        ],
        out_specs=[
            pl.BlockSpec((bm, dz), row),
            pl.BlockSpec((bm, dz), row),
        ],
        out_shape=[
            jax.ShapeDtypeStruct((n, dz), jnp.float32),
            jax.ShapeDtypeStruct((n, dz), jnp.float32),
        ],
    )(acc2, hs2, dist, bc)

    bn = 1280
    gj = pl.cdiv(n, bn)
    adj = pl.pallas_call(
        _tc_adj_body,
        grid=(gi, gj),
        in_specs=[
            pl.BlockSpec((bm, dz), lambda i, j: (i, 0)),
            pl.BlockSpec((bn, dz), lambda i, j: (j, 0)),
        ],
        out_specs=pl.BlockSpec((bm, bn), lambda i, j: (i, j)),
        out_shape=jax.ShapeDtypeStruct((n, n), jnp.float32),
    )(mu, mu)

    return (mu, logvar, mu, adj)


# speedup vs baseline: 19.4061x; 1.1266x over previous
import functools

import jax
import jax.numpy as jnp
from jax import lax
from jax.experimental import pallas as pl
from jax.experimental.pallas import tpu as pltpu
from jax.experimental.pallas import tpu_sc as plsc

NC, NS, LANES = 2, 16, 16
NW = NC * NS


def _fill_rows(ref, rows, d, val):
    vec = jnp.full((LANES,), val, jnp.float32)

    def body(r, carry):
        for j in range(d // LANES):
            ref[r, pl.ds(j * LANES, LANES)] = vec
        return carry

    lax.fori_loop(0, rows, body, 0)


def _chunking(per_w):
    k = 80 if per_w % 80 == 0 else 8
    return k, per_w // k


def _slabbing(nchunk):
    h = 5 if nchunk % 5 == 0 else 1
    return h, nchunk // h


def _make_sc_scatter(n, e, d):
    per_w = e // NW
    k, nchunk = _chunking(per_w)
    nh, snc = _slabbing(nchunk)
    n_pad = ((n + 127) // 128) * 128
    rpt = n_pad // NS

    mesh = plsc.VectorSubcoreMesh(core_axis_name="c", subcore_axis_name="s")

    zr = 8

    scratch = [
        pltpu.VMEM((snc, k), jnp.int32),
        pltpu.VMEM((snc, k), jnp.int32),
        pltpu.VMEM((k, d), jnp.float32),
        pltpu.VMEM((k, d), jnp.float32),
        pltpu.VMEM((zr, d), jnp.float32),
        pltpu.VMEM_SHARED((n_pad, d), jnp.float32),
        pltpu.SemaphoreType.DMA,
        pltpu.SemaphoreType.DMA,
    ]

    def body(vals_hbm, src_hbm, dst_hbm, out_hbm,
             src_v, dst_v, rows0, rows1, zbuf, acc, sem0, sem1):
        c = lax.axis_index("c")
        s = lax.axis_index("s")
        wid = c * NS + s

        _fill_rows(zbuf, zr, d, 0.0)
        r0 = s * rpt

        def zero(i, carry):
            pltpu.sync_copy(zbuf, acc.at[pl.ds(r0 + i * zr, zr)])
            return carry

        lax.fori_loop(0, rpt // zr, zero, 0)
        plsc.subcore_barrier()

        def pair(i, carry):
            j = 2 * i
            h0 = pltpu.async_copy(vals_hbm.at[src_v.at[j]], rows0, sem0)
            h1 = pltpu.async_copy(vals_hbm.at[src_v.at[j + 1]], rows1, sem1)
            h0.wait()
            pltpu.sync_copy(rows0, acc.at[dst_v.at[j]], add=True)
            h1.wait()
            pltpu.sync_copy(rows1, acc.at[dst_v.at[j + 1]], add=True)
            return carry

        def slab(h, carry):
            pltpu.sync_copy(src_hbm.at[wid].at[h], src_v)
            pltpu.sync_copy(dst_hbm.at[wid].at[h], dst_v)
            lax.fori_loop(0, snc // 2, pair, 0)
            for j in range(2 * (snc // 2), snc):
                pltpu.async_copy(vals_hbm.at[src_v.at[j]], rows0, sem0).wait()
                pltpu.sync_copy(rows0, acc.at[dst_v.at[j]], add=True)
            return carry

        lax.fori_loop(0, nh, slab, 0)
        plsc.subcore_barrier()

        pltpu.sync_copy(acc.at[pl.ds(r0, rpt)], out_hbm.at[c].at[pl.ds(r0, rpt)])

    return pl.kernel(
        body,
        out_type=jax.ShapeDtypeStruct((NC, n_pad, d), jnp.float32),
        mesh=mesh,
        scratch_types=scratch,
    )


def _make_sc_degree(n, e):
    per_w = e // NW
    k, nchunk = _chunking(per_w)
    n_pad = ((n + 2047) // 2048) * 2048
    rpt = n_pad // NS

    mesh = plsc.VectorSubcoreMesh(core_axis_name="c", subcore_axis_name="s")

    scratch = [
        pltpu.VMEM((nchunk, k), jnp.int32),
        pltpu.VMEM((k,), jnp.float32),
        pltpu.VMEM((rpt,), jnp.float32),
        pltpu.VMEM_SHARED((n_pad,), jnp.float32),
        pltpu.SemaphoreType.DMA,
    ]

    def body(dst_hbm, out_hbm, dst_v, ones_v, zbuf, acc, sem):
        c = lax.axis_index("c")
        s = lax.axis_index("s")
        wid = c * NS + s

        pltpu.sync_copy(dst_hbm.at[wid], dst_v)

        def fill1(r, carry):
            ones_v[pl.ds(r * LANES, LANES)] = jnp.ones((LANES,), jnp.float32)
            return carry
        lax.fori_loop(0, k // LANES, fill1, 0)

        def fill0(r, carry):
            zbuf[pl.ds(r * LANES, LANES)] = jnp.zeros((LANES,), jnp.float32)
            return carry
        lax.fori_loop(0, rpt // LANES, fill0, 0)

        r0 = s * rpt
        pltpu.sync_copy(zbuf, acc.at[pl.ds(r0, rpt)])
        plsc.subcore_barrier()

        def chunk(j, carry):
            pltpu.sync_copy(ones_v, acc.at[dst_v.at[j]], add=True)
            return carry

        lax.fori_loop(0, nchunk, chunk, 0)
        plsc.subcore_barrier()

        pltpu.sync_copy(acc.at[pl.ds(r0, rpt)], zbuf)
        pltpu.sync_copy(zbuf, out_hbm.at[c].at[pl.ds(r0, rpt)])

    return pl.kernel(
        body,
        out_type=jax.ShapeDtypeStruct((NC, n_pad), jnp.float32),
        mesh=mesh,
        scratch_types=scratch,
    )


def _rowscale(xblk, dis_t):
    g = dis_t.shape[1]
    parts = [xblk[128 * i:128 * (i + 1), :] * dis_t[:, i:i + 1] for i in range(g)]
    return jnp.concatenate(parts, axis=0)


def _tc_a_body(x_ref, w1_ref, degp_ref, hs1_ref, dist_ref):
    deg = 1.0 + degp_ref[0] + degp_ref[1]
    dis_t = jnp.transpose(1.0 / jnp.sqrt(deg))
    dist_ref[0] = dis_t
    xw = jnp.dot(x_ref[...], w1_ref[...], preferred_element_type=jnp.float32)
    hs1_ref[...] = _rowscale(xw, dis_t)


def _tc_b_body(a_ref, hs1_ref, dist_ref, b1_ref, wc_ref, o_ref, dh):
    dis_t = dist_ref[0]
    agg = a_ref[0] + a_ref[1] + hs1_ref[...]
    h = jnp.maximum(_rowscale(agg, dis_t) + b1_ref[...], 0.0)
    hw = jnp.dot(h, wc_ref[...], preferred_element_type=jnp.float32)
    hs2 = _rowscale(hw, dis_t)
    o_ref[...] = jnp.concatenate([hs2] * (dh // hs2.shape[1]), axis=1)


def _tc_c_body(a_ref, hs2_ref, dist_ref, bc_ref, mu_ref, lv_ref, dz):
    dis_t = dist_ref[0]
    agg = a_ref[0] + a_ref[1]
    mu_ref[...] = _rowscale(agg[:, :dz] + hs2_ref[:, :dz], dis_t) + bc_ref[:, :dz]
    lv_ref[...] = _rowscale(agg[:, dz:2 * dz] + hs2_ref[:, dz:2 * dz],
                            dis_t) + bc_ref[:, dz:]


def _tc_adj_body(zi_ref, zj_ref, o_ref):
    logits = lax.dot_general(
        zi_ref[...], zj_ref[...], (((1,), (1,)), ((), ())),
        preferred_element_type=jnp.float32)
    o_ref[...] = jax.nn.sigmoid(logits)


def kernel(x, edge_index, W1, b1, Wmu, bmu, Wlv, blv):
    n, din = x.shape
    e = edge_index.shape[1]
    dh = W1.shape[1]
    dz = Wmu.shape[1]
    d2 = 2 * dz

    k, nchunk = _chunking(e // NW)
    nh, snc = _slabbing(nchunk)
    src_w = edge_index[0].reshape(NW, nh, snc, k)
    dst_w = edge_index[1].reshape(NW, nh, snc, k)
    dst_flat = edge_index[1].reshape(NW, nchunk, k)

    wc = jnp.concatenate([Wmu, Wlv], axis=1)
    bc = jnp.concatenate([bmu, blv]).reshape(1, d2)
    b1r = b1.reshape(1, dh)

    n_pad_deg = ((n + 2047) // 2048) * 2048

    degp = _make_sc_degree(n, e)(dst_flat)
    degp = degp.reshape(NC, n_pad_deg // 128, 128)

    bm = 1024
    g = bm // 128
    gi = pl.cdiv(n, bm)
    row = lambda i: (i, 0)
    halfrow = lambda i: (0, i, 0)

    hs1, dist = pl.pallas_call(
        _tc_a_body,
        grid=(gi,),
        in_specs=[
            pl.BlockSpec((bm, din), row),
            pl.BlockSpec((din, dh), lambda i: (0, 0)),
            pl.BlockSpec((NC, g, 128), halfrow),
        ],
        out_specs=[
            pl.BlockSpec((bm, dh), row),
            pl.BlockSpec((1, 128, g), lambda i: (i, 0, 0)),
        ],
        out_shape=[
            jax.ShapeDtypeStruct((n, dh), jnp.float32),
            jax.ShapeDtypeStruct((gi, 128, g), jnp.float32),
        ],
    )(x, W1, degp)

    acc1 = _make_sc_scatter(n, e, dh)(hs1, src_w, dst_w)

    hs2 = pl.pallas_call(
        functools.partial(_tc_b_body, dh=dh),
        grid=(gi,),
        in_specs=[
            pl.BlockSpec((NC, bm, dh), halfrow),
            pl.BlockSpec((bm, dh), row),
            pl.BlockSpec((1, 128, g), lambda i: (i, 0, 0)),
            pl.BlockSpec((1, dh), lambda i: (0, 0)),
            pl.BlockSpec((dh, d2), lambda i: (0, 0)),
        ],
        out_specs=pl.BlockSpec((bm, dh), row),
        out_shape=jax.ShapeDtypeStruct((n, dh), jnp.float32),
    )(acc1, hs1, dist, b1r, wc)

    acc2 = _make_sc_scatter(n, e, dh)(hs2, src_w, dst_w)

    mu, logvar = pl.pallas_call(
        functools.partial(_tc_c_body, dz=dz),
        grid=(gi,),
        in_specs=[
            pl.BlockSpec((NC, bm, dh), halfrow),
            pl.BlockSpec((bm, dh), row),
            pl.BlockSpec((1, 128, g), lambda i: (i, 0, 0)),
            pl.BlockSpec((1, d2), lambda i: (0, 0)),
        ],
        out_specs=[
            pl.BlockSpec((bm, dz), row),
            pl.BlockSpec((bm, dz), row),
        ],
        out_shape=[
            jax.ShapeDtypeStruct((n, dz), jnp.float32),
            jax.ShapeDtypeStruct((n, dz), jnp.float32),
        ],
    )(acc2, hs2, dist, bc)

    bn = 1280
    gj = pl.cdiv(n, bn)
    adj = pl.pallas_call(
        _tc_adj_body,
        grid=(gi, gj),
        in_specs=[
            pl.BlockSpec((bm, dz), lambda i, j: (i, 0)),
            pl.BlockSpec((bn, dz), lambda i, j: (j, 0)),
        ],
        out_specs=pl.BlockSpec((bm, bn), lambda i, j: (i, j)),
        out_shape=jax.ShapeDtypeStruct((n, n), jnp.float32),
    )(mu, mu)

    return (mu, logvar, mu, adj)


# speedup vs baseline: 20.1640x; 1.0391x over previous
import functools

import jax
import jax.numpy as jnp
from jax import lax
from jax.experimental import pallas as pl
from jax.experimental.pallas import tpu as pltpu
from jax.experimental.pallas import tpu_sc as plsc

NC, NS, LANES = 2, 16, 16
NW = NC * NS


def _fill_rows(ref, rows, d, val):
    vec = jnp.full((LANES,), val, jnp.float32)

    def body(r, carry):
        for j in range(d // LANES):
            ref[r, pl.ds(j * LANES, LANES)] = vec
        return carry

    lax.fori_loop(0, rows, body, 0)


def _chunking(per_w):
    k = 80 if per_w % 80 == 0 else 8
    return k, per_w // k


def _slabbing(nchunk):
    h = 5 if nchunk % 5 == 0 else 1
    return h, nchunk // h


def _make_sc_scatter(n, e, d):
    per_w = e // NW
    k, nchunk = _chunking(per_w)
    nh, snc = _slabbing(nchunk)
    n_pad = ((n + 127) // 128) * 128
    rpt = n_pad // NS

    mesh = plsc.VectorSubcoreMesh(core_axis_name="c", subcore_axis_name="s")

    zr = 8

    nb = 4

    scratch = [
        pltpu.VMEM((snc, k), jnp.int32),
        pltpu.VMEM((snc, k), jnp.int32),
    ] + [pltpu.VMEM((k, d), jnp.float32)] * nb + [
        pltpu.VMEM((zr, d), jnp.float32),
        pltpu.VMEM_SHARED((n_pad, d), jnp.float32),
    ] + [pltpu.SemaphoreType.DMA] * nb

    def body(vals_hbm, src_hbm, dst_hbm, out_hbm, src_v, dst_v, *rest):
        rows = rest[:nb]
        zbuf, acc = rest[nb], rest[nb + 1]
        sems = rest[nb + 2:]
        c = lax.axis_index("c")
        s = lax.axis_index("s")
        wid = c * NS + s

        _fill_rows(zbuf, zr, d, 0.0)
        r0 = s * rpt

        def zero(i, carry):
            pltpu.sync_copy(zbuf, acc.at[pl.ds(r0 + i * zr, zr)])
            return carry

        lax.fori_loop(0, rpt // zr, zero, 0)
        plsc.subcore_barrier()

        def group(i, carry):
            j = nb * i
            hs = [pltpu.async_copy(vals_hbm.at[src_v.at[j + t]], rows[t], sems[t])
                  for t in range(nb)]
            for t in range(nb):
                hs[t].wait()
                pltpu.sync_copy(rows[t], acc.at[dst_v.at[j + t]], add=True)
            return carry

        def slab(h, carry):
            pltpu.sync_copy(src_hbm.at[wid].at[h], src_v)
            pltpu.sync_copy(dst_hbm.at[wid].at[h], dst_v)
            lax.fori_loop(0, snc // nb, group, 0)
            for j in range(nb * (snc // nb), snc):
                t = j % nb
                pltpu.async_copy(vals_hbm.at[src_v.at[j]], rows[t], sems[t]).wait()
                pltpu.sync_copy(rows[t], acc.at[dst_v.at[j]], add=True)
            return carry

        lax.fori_loop(0, nh, slab, 0)
        plsc.subcore_barrier()

        pltpu.sync_copy(acc.at[pl.ds(r0, rpt)], out_hbm.at[c].at[pl.ds(r0, rpt)])

    return pl.kernel(
        body,
        out_type=jax.ShapeDtypeStruct((NC, n_pad, d), jnp.float32),
        mesh=mesh,
        scratch_types=scratch,
    )


def _make_sc_degree(n, e):
    per_w = e // NW
    k, nchunk = _chunking(per_w)
    n_pad = ((n + 2047) // 2048) * 2048
    rpt = n_pad // NS

    mesh = plsc.VectorSubcoreMesh(core_axis_name="c", subcore_axis_name="s")

    scratch = [
        pltpu.VMEM((nchunk, k), jnp.int32),
        pltpu.VMEM((k,), jnp.float32),
        pltpu.VMEM((rpt,), jnp.float32),
        pltpu.VMEM_SHARED((n_pad,), jnp.float32),
        pltpu.SemaphoreType.DMA,
    ]

    def body(dst_hbm, out_hbm, dst_v, ones_v, zbuf, acc, sem):
        c = lax.axis_index("c")
        s = lax.axis_index("s")
        wid = c * NS + s

        pltpu.sync_copy(dst_hbm.at[wid], dst_v)

        def fill1(r, carry):
            ones_v[pl.ds(r * LANES, LANES)] = jnp.ones((LANES,), jnp.float32)
            return carry
        lax.fori_loop(0, k // LANES, fill1, 0)

        def fill0(r, carry):
            zbuf[pl.ds(r * LANES, LANES)] = jnp.zeros((LANES,), jnp.float32)
            return carry
        lax.fori_loop(0, rpt // LANES, fill0, 0)

        r0 = s * rpt
        pltpu.sync_copy(zbuf, acc.at[pl.ds(r0, rpt)])
        plsc.subcore_barrier()

        def chunk(j, carry):
            pltpu.sync_copy(ones_v, acc.at[dst_v.at[j]], add=True)
            return carry

        lax.fori_loop(0, nchunk, chunk, 0)
        plsc.subcore_barrier()

        pltpu.sync_copy(acc.at[pl.ds(r0, rpt)], zbuf)
        pltpu.sync_copy(zbuf, out_hbm.at[c].at[pl.ds(r0, rpt)])

    return pl.kernel(
        body,
        out_type=jax.ShapeDtypeStruct((NC, n_pad), jnp.float32),
        mesh=mesh,
        scratch_types=scratch,
    )


def _rowscale(xblk, dis_t):
    g = dis_t.shape[1]
    parts = [xblk[128 * i:128 * (i + 1), :] * dis_t[:, i:i + 1] for i in range(g)]
    return jnp.concatenate(parts, axis=0)


def _tc_a_body(x_ref, w1_ref, degp_ref, hs1_ref, dist_ref):
    deg = 1.0 + degp_ref[0] + degp_ref[1]
    dis_t = jnp.transpose(1.0 / jnp.sqrt(deg))
    dist_ref[0] = dis_t
    xw = jnp.dot(x_ref[...], w1_ref[...], preferred_element_type=jnp.float32)
    hs1_ref[...] = _rowscale(xw, dis_t)


def _tc_b_body(a_ref, hs1_ref, dist_ref, b1_ref, wc_ref, o_ref, dh):
    dis_t = dist_ref[0]
    agg = a_ref[0] + a_ref[1] + hs1_ref[...]
    h = jnp.maximum(_rowscale(agg, dis_t) + b1_ref[...], 0.0)
    hw = jnp.dot(h, wc_ref[...], preferred_element_type=jnp.float32)
    hs2 = _rowscale(hw, dis_t)
    o_ref[...] = jnp.concatenate([hs2] * (dh // hs2.shape[1]), axis=1)


def _tc_c_body(a_ref, hs2_ref, dist_ref, bc_ref, mu_ref, lv_ref, dz):
    dis_t = dist_ref[0]
    agg = a_ref[0] + a_ref[1]
    mu_ref[...] = _rowscale(agg[:, :dz] + hs2_ref[:, :dz], dis_t) + bc_ref[:, :dz]
    lv_ref[...] = _rowscale(agg[:, dz:2 * dz] + hs2_ref[:, dz:2 * dz],
                            dis_t) + bc_ref[:, dz:]


def _tc_adj_body(zi_ref, zj_ref, o_ref):
    logits = lax.dot_general(
        zi_ref[...], zj_ref[...], (((1,), (1,)), ((), ())),
        preferred_element_type=jnp.float32)
    o_ref[...] = jax.nn.sigmoid(logits)


def kernel(x, edge_index, W1, b1, Wmu, bmu, Wlv, blv):
    n, din = x.shape
    e = edge_index.shape[1]
    dh = W1.shape[1]
    dz = Wmu.shape[1]
    d2 = 2 * dz

    k, nchunk = _chunking(e // NW)
    nh, snc = _slabbing(nchunk)
    src_w = edge_index[0].reshape(NW, nh, snc, k)
    dst_w = edge_index[1].reshape(NW, nh, snc, k)
    dst_flat = edge_index[1].reshape(NW, nchunk, k)

    wc = jnp.concatenate([Wmu, Wlv], axis=1)
    bc = jnp.concatenate([bmu, blv]).reshape(1, d2)
    b1r = b1.reshape(1, dh)

    n_pad_deg = ((n + 2047) // 2048) * 2048

    degp = _make_sc_degree(n, e)(dst_flat)
    degp = degp.reshape(NC, n_pad_deg // 128, 128)

    bm = 1024
    g = bm // 128
    gi = pl.cdiv(n, bm)
    row = lambda i: (i, 0)
    halfrow = lambda i: (0, i, 0)

    hs1, dist = pl.pallas_call(
        _tc_a_body,
        grid=(gi,),
        in_specs=[
            pl.BlockSpec((bm, din), row),
            pl.BlockSpec((din, dh), lambda i: (0, 0)),
            pl.BlockSpec((NC, g, 128), halfrow),
        ],
        out_specs=[
            pl.BlockSpec((bm, dh), row),
            pl.BlockSpec((1, 128, g), lambda i: (i, 0, 0)),
        ],
        out_shape=[
            jax.ShapeDtypeStruct((n, dh), jnp.float32),
            jax.ShapeDtypeStruct((gi, 128, g), jnp.float32),
        ],
    )(x, W1, degp)

    acc1 = _make_sc_scatter(n, e, dh)(hs1, src_w, dst_w)

    hs2 = pl.pallas_call(
        functools.partial(_tc_b_body, dh=dh),
        grid=(gi,),
        in_specs=[
            pl.BlockSpec((NC, bm, dh), halfrow),
            pl.BlockSpec((bm, dh), row),
            pl.BlockSpec((1, 128, g), lambda i: (i, 0, 0)),
            pl.BlockSpec((1, dh), lambda i: (0, 0)),
            pl.BlockSpec((dh, d2), lambda i: (0, 0)),
        ],
        out_specs=pl.BlockSpec((bm, dh), row),
        out_shape=jax.ShapeDtypeStruct((n, dh), jnp.float32),
    )(acc1, hs1, dist, b1r, wc)

    acc2 = _make_sc_scatter(n, e, dh)(hs2, src_w, dst_w)

    mu, logvar = pl.pallas_call(
        functools.partial(_tc_c_body, dz=dz),
        grid=(gi,),
        in_specs=[
            pl.BlockSpec((NC, bm, dh), halfrow),
            pl.BlockSpec((bm, dh), row),
            pl.BlockSpec((1, 128, g), lambda i: (i, 0, 0)),
            pl.BlockSpec((1, d2), lambda i: (0, 0)),
        ],
        out_specs=[
            pl.BlockSpec((bm, dz), row),
            pl.BlockSpec((bm, dz), row),
        ],
        out_shape=[
            jax.ShapeDtypeStruct((n, dz), jnp.float32),
            jax.ShapeDtypeStruct((n, dz), jnp.float32),
        ],
    )(acc2, hs2, dist, bc)

    bn = 1280
    gj = pl.cdiv(n, bn)
    adj = pl.pallas_call(
        _tc_adj_body,
        grid=(gi, gj),
        in_specs=[
            pl.BlockSpec((bm, dz), lambda i, j: (i, 0)),
            pl.BlockSpec((bn, dz), lambda i, j: (j, 0)),
        ],
        out_specs=pl.BlockSpec((bm, bn), lambda i, j: (i, j)),
        out_shape=jax.ShapeDtypeStruct((n, n), jnp.float32),
    )(mu, mu)

    return (mu, logvar, mu, adj)


# speedup vs baseline: 20.1936x; 1.0015x over previous
import functools

import jax
import jax.numpy as jnp
from jax import lax
from jax.experimental import pallas as pl
from jax.experimental.pallas import tpu as pltpu
from jax.experimental.pallas import tpu_sc as plsc

NC, NS, LANES = 2, 16, 16
NW = NC * NS


def _fill_rows(ref, rows, d, val):
    vec = jnp.full((LANES,), val, jnp.float32)

    def body(r, carry):
        for j in range(d // LANES):
            ref[r, pl.ds(j * LANES, LANES)] = vec
        return carry

    lax.fori_loop(0, rows, body, 0)


def _chunking(per_w):
    k = 80 if per_w % 80 == 0 else 8
    return k, per_w // k


def _slabbing(nchunk):
    h = 5 if nchunk % 5 == 0 else 1
    return h, nchunk // h


def _make_sc_scatter(n, e, d):
    per_w = e // NW
    k, nchunk = _chunking(per_w)
    nh, snc = _slabbing(nchunk)
    n_pad = ((n + 127) // 128) * 128
    rpt = n_pad // NS

    mesh = plsc.VectorSubcoreMesh(core_axis_name="c", subcore_axis_name="s")

    zr = 8

    nb = 4

    scratch = [
        pltpu.VMEM((snc, k), jnp.int32),
        pltpu.VMEM((snc, k), jnp.int32),
    ] + [pltpu.VMEM((k, d), jnp.float32)] * nb + [
        pltpu.VMEM((zr, d), jnp.float32),
        pltpu.VMEM_SHARED((n_pad, d), jnp.float32),
    ] + [pltpu.SemaphoreType.DMA] * nb

    def body(vals_hbm, src_hbm, dst_hbm, out_hbm, src_v, dst_v, *rest):
        rows = rest[:nb]
        zbuf, acc = rest[nb], rest[nb + 1]
        sems = rest[nb + 2:]
        c = lax.axis_index("c")
        s = lax.axis_index("s")
        wid = c * NS + s

        _fill_rows(zbuf, zr, d, 0.0)
        r0 = s * rpt

        def zero(i, carry):
            pltpu.sync_copy(zbuf, acc.at[pl.ds(r0 + i * zr, zr)])
            return carry

        lax.fori_loop(0, rpt // zr, zero, 0)
        plsc.subcore_barrier()

        def group(i, carry):
            j = nb * i
            hs = [pltpu.async_copy(vals_hbm.at[src_v.at[j + t]], rows[t], sems[t])
                  for t in range(nb)]
            for t in range(nb):
                hs[t].wait()
                pltpu.sync_copy(rows[t], acc.at[dst_v.at[j + t]], add=True)
            return carry

        def slab(h, carry):
            pltpu.sync_copy(src_hbm.at[wid].at[h], src_v)
            pltpu.sync_copy(dst_hbm.at[wid].at[h], dst_v)
            lax.fori_loop(0, snc // nb, group, 0)
            for j in range(nb * (snc // nb), snc):
                t = j % nb
                pltpu.async_copy(vals_hbm.at[src_v.at[j]], rows[t], sems[t]).wait()
                pltpu.sync_copy(rows[t], acc.at[dst_v.at[j]], add=True)
            return carry

        lax.fori_loop(0, nh, slab, 0)
        plsc.subcore_barrier()

        pltpu.sync_copy(acc.at[pl.ds(r0, rpt)], out_hbm.at[c].at[pl.ds(r0, rpt)])

    return pl.kernel(
        body,
        out_type=jax.ShapeDtypeStruct((NC, n_pad, d), jnp.float32),
        mesh=mesh,
        scratch_types=scratch,
    )


def _make_sc_degree(n, e):
    per_w = e // NW
    k, nchunk = _chunking(per_w)
    n_pad = ((n + 2047) // 2048) * 2048
    rpt = n_pad // NS

    mesh = plsc.VectorSubcoreMesh(core_axis_name="c", subcore_axis_name="s")

    scratch = [
        pltpu.VMEM((nchunk, k), jnp.int32),
        pltpu.VMEM((k,), jnp.float32),
        pltpu.VMEM((rpt,), jnp.float32),
        pltpu.VMEM_SHARED((n_pad,), jnp.float32),
        pltpu.SemaphoreType.DMA,
    ]

    def body(dst_hbm, out_hbm, dst_v, ones_v, zbuf, acc, sem):
        c = lax.axis_index("c")
        s = lax.axis_index("s")
        wid = c * NS + s

        pltpu.sync_copy(dst_hbm.at[wid], dst_v)

        def fill1(r, carry):
            ones_v[pl.ds(r * LANES, LANES)] = jnp.ones((LANES,), jnp.float32)
            return carry
        lax.fori_loop(0, k // LANES, fill1, 0)

        def fill0(r, carry):
            zbuf[pl.ds(r * LANES, LANES)] = jnp.zeros((LANES,), jnp.float32)
            return carry
        lax.fori_loop(0, rpt // LANES, fill0, 0)

        r0 = s * rpt
        pltpu.sync_copy(zbuf, acc.at[pl.ds(r0, rpt)])
        plsc.subcore_barrier()

        def chunk(j, carry):
            pltpu.sync_copy(ones_v, acc.at[dst_v.at[j]], add=True)
            return carry

        lax.fori_loop(0, nchunk, chunk, 0)
        plsc.subcore_barrier()

        pltpu.sync_copy(acc.at[pl.ds(r0, rpt)], zbuf)
        pltpu.sync_copy(zbuf, out_hbm.at[c].at[pl.ds(r0, rpt)])

    return pl.kernel(
        body,
        out_type=jax.ShapeDtypeStruct((NC, n_pad), jnp.float32),
        mesh=mesh,
        scratch_types=scratch,
    )


def _rowscale(xblk, dis_t):
    g = dis_t.shape[1]
    parts = [xblk[128 * i:128 * (i + 1), :] * dis_t[:, i:i + 1] for i in range(g)]
    return jnp.concatenate(parts, axis=0)


def _tc_a0_body(x_ref, w1_ref, o_ref):
    o_ref[...] = jnp.dot(x_ref[...], w1_ref[...],
                         preferred_element_type=jnp.float32)


def _tc_a1_body(xw_ref, degp_ref, hs1_ref, dist_ref):
    deg = 1.0 + degp_ref[0] + degp_ref[1]
    dis_t = jnp.transpose(1.0 / jnp.sqrt(deg))
    dist_ref[0] = dis_t
    hs1_ref[...] = _rowscale(xw_ref[...], dis_t)


def _tc_b_body(a_ref, hs1_ref, dist_ref, b1_ref, wc_ref, o_ref, dh):
    dis_t = dist_ref[0]
    agg = a_ref[0] + a_ref[1] + hs1_ref[...]
    h = jnp.maximum(_rowscale(agg, dis_t) + b1_ref[...], 0.0)
    hw = jnp.dot(h, wc_ref[...], preferred_element_type=jnp.float32)
    hs2 = _rowscale(hw, dis_t)
    o_ref[...] = jnp.concatenate([hs2] * (dh // hs2.shape[1]), axis=1)


def _tc_c_body(a_ref, hs2_ref, dist_ref, bc_ref, mu_ref, lv_ref, dz):
    dis_t = dist_ref[0]
    agg = a_ref[0] + a_ref[1]
    mu_ref[...] = _rowscale(agg[:, :dz] + hs2_ref[:, :dz], dis_t) + bc_ref[:, :dz]
    lv_ref[...] = _rowscale(agg[:, dz:2 * dz] + hs2_ref[:, dz:2 * dz],
                            dis_t) + bc_ref[:, dz:]


def _tc_adj_body(zi_ref, zj_ref, o_ref):
    logits = lax.dot_general(
        zi_ref[...], zj_ref[...], (((1,), (1,)), ((), ())),
        preferred_element_type=jnp.float32)
    o_ref[...] = jax.nn.sigmoid(logits)


def kernel(x, edge_index, W1, b1, Wmu, bmu, Wlv, blv):
    n, din = x.shape
    e = edge_index.shape[1]
    dh = W1.shape[1]
    dz = Wmu.shape[1]
    d2 = 2 * dz

    k, nchunk = _chunking(e // NW)
    nh, snc = _slabbing(nchunk)
    src_w = edge_index[0].reshape(NW, nh, snc, k)
    dst_w = edge_index[1].reshape(NW, nh, snc, k)
    dst_flat = edge_index[1].reshape(NW, nchunk, k)

    wc = jnp.concatenate([Wmu, Wlv], axis=1)
    bc = jnp.concatenate([bmu, blv]).reshape(1, d2)
    b1r = b1.reshape(1, dh)

    n_pad_deg = ((n + 2047) // 2048) * 2048

    degp = _make_sc_degree(n, e)(dst_flat)
    degp = degp.reshape(NC, n_pad_deg // 128, 128)

    bm = 1024
    g = bm // 128
    gi = pl.cdiv(n, bm)
    row = lambda i: (i, 0)
    halfrow = lambda i: (0, i, 0)

    xw = pl.pallas_call(
        _tc_a0_body,
        grid=(gi,),
        in_specs=[
            pl.BlockSpec((bm, din), row),
            pl.BlockSpec((din, dh), lambda i: (0, 0)),
        ],
        out_specs=pl.BlockSpec((bm, dh), row),
        out_shape=jax.ShapeDtypeStruct((n, dh), jnp.float32),
    )(x, W1)

    hs1, dist = pl.pallas_call(
        _tc_a1_body,
        grid=(gi,),
        in_specs=[
            pl.BlockSpec((bm, dh), row),
            pl.BlockSpec((NC, g, 128), halfrow),
        ],
        out_specs=[
            pl.BlockSpec((bm, dh), row),
            pl.BlockSpec((1, 128, g), lambda i: (i, 0, 0)),
        ],
        out_shape=[
            jax.ShapeDtypeStruct((n, dh), jnp.float32),
            jax.ShapeDtypeStruct((gi, 128, g), jnp.float32),
        ],
    )(xw, degp)

    acc1 = _make_sc_scatter(n, e, dh)(hs1, src_w, dst_w)

    hs2 = pl.pallas_call(
        functools.partial(_tc_b_body, dh=dh),
        grid=(gi,),
        in_specs=[
            pl.BlockSpec((NC, bm, dh), halfrow),
            pl.BlockSpec((bm, dh), row),
            pl.BlockSpec((1, 128, g), lambda i: (i, 0, 0)),
            pl.BlockSpec((1, dh), lambda i: (0, 0)),
            pl.BlockSpec((dh, d2), lambda i: (0, 0)),
        ],
        out_specs=pl.BlockSpec((bm, dh), row),
        out_shape=jax.ShapeDtypeStruct((n, dh), jnp.float32),
    )(acc1, hs1, dist, b1r, wc)

    acc2 = _make_sc_scatter(n, e, dh)(hs2, src_w, dst_w)

    mu, logvar = pl.pallas_call(
        functools.partial(_tc_c_body, dz=dz),
        grid=(gi,),
        in_specs=[
            pl.BlockSpec((NC, bm, dh), halfrow),
            pl.BlockSpec((bm, dh), row),
            pl.BlockSpec((1, 128, g), lambda i: (i, 0, 0)),
            pl.BlockSpec((1, d2), lambda i: (0, 0)),
        ],
        out_specs=[
            pl.BlockSpec((bm, dz), row),
            pl.BlockSpec((bm, dz), row),
        ],
        out_shape=[
            jax.ShapeDtypeStruct((n, dz), jnp.float32),
            jax.ShapeDtypeStruct((n, dz), jnp.float32),
        ],
    )(acc2, hs2, dist, bc)

    bn = 1280
    gj = pl.cdiv(n, bn)
    adj = pl.pallas_call(
        _tc_adj_body,
        grid=(gi, gj),
        in_specs=[
            pl.BlockSpec((bm, dz), lambda i, j: (i, 0)),
            pl.BlockSpec((bn, dz), lambda i, j: (j, 0)),
        ],
        out_specs=pl.BlockSpec((bm, bn), lambda i, j: (i, j)),
        out_shape=jax.ShapeDtypeStruct((n, n), jnp.float32),
    )(mu, mu)

    return (mu, logvar, mu, adj)


# speedup vs baseline: 21.3338x; 1.0565x over previous
import functools

import jax
import jax.numpy as jnp
from jax import lax
from jax.experimental import pallas as pl
from jax.experimental.pallas import tpu as pltpu
from jax.experimental.pallas import tpu_sc as plsc

NC, NS, LANES = 2, 16, 16
NW = NC * NS


def _fill_rows(ref, rows, d, val):
    vec = jnp.full((LANES,), val, jnp.float32)

    def body(r, carry):
        for j in range(d // LANES):
            ref[r, pl.ds(j * LANES, LANES)] = vec
        return carry

    lax.fori_loop(0, rows, body, 0)


def _chunking(per_w):
    k = 80 if per_w % 80 == 0 else 8
    return k, per_w // k


def _slabbing(nchunk):
    h = 5 if nchunk % 5 == 0 else 1
    return h, nchunk // h


def _make_sc_scatter(n, e, d):
    per_w = e // NW
    k, nchunk = _chunking(per_w)
    nh, snc = _slabbing(nchunk)
    n_pad = ((n + 127) // 128) * 128
    rpt = n_pad // NS

    mesh = plsc.VectorSubcoreMesh(core_axis_name="c", subcore_axis_name="s")

    zr = 8

    nb = 4

    scratch = [
        pltpu.VMEM((snc, k), jnp.int32),
        pltpu.VMEM((snc, k), jnp.int32),
    ] + [pltpu.VMEM((k, d), jnp.float32)] * nb + [
        pltpu.VMEM((zr, d), jnp.float32),
        pltpu.VMEM_SHARED((n_pad, d), jnp.float32),
    ] + [pltpu.SemaphoreType.DMA] * nb

    def body(vals_hbm, src_hbm, dst_hbm, out_hbm, src_v, dst_v, *rest):
        rows = rest[:nb]
        zbuf, acc = rest[nb], rest[nb + 1]
        sems = rest[nb + 2:]
        c = lax.axis_index("c")
        s = lax.axis_index("s")
        wid = c * NS + s

        _fill_rows(zbuf, zr, d, 0.0)
        r0 = s * rpt

        def zero(i, carry):
            pltpu.sync_copy(zbuf, acc.at[pl.ds(r0 + i * zr, zr)])
            return carry

        lax.fori_loop(0, rpt // zr, zero, 0)
        plsc.subcore_barrier()

        def group(i, carry):
            j = nb * i
            hs = [pltpu.async_copy(vals_hbm.at[src_v.at[j + t]], rows[t], sems[t])
                  for t in range(nb)]
            for t in range(nb):
                hs[t].wait()
                pltpu.sync_copy(rows[t], acc.at[dst_v.at[j + t]], add=True)
            return carry

        def slab(h, carry):
            pltpu.sync_copy(src_hbm.at[wid].at[h], src_v)
            pltpu.sync_copy(dst_hbm.at[wid].at[h], dst_v)
            lax.fori_loop(0, snc // nb, group, 0)
            for j in range(nb * (snc // nb), snc):
                t = j % nb
                pltpu.async_copy(vals_hbm.at[src_v.at[j]], rows[t], sems[t]).wait()
                pltpu.sync_copy(rows[t], acc.at[dst_v.at[j]], add=True)
            return carry

        lax.fori_loop(0, nh, slab, 0)
        plsc.subcore_barrier()

        pltpu.sync_copy(acc.at[pl.ds(r0, rpt)], out_hbm.at[c].at[pl.ds(r0, rpt)])

    return pl.kernel(
        body,
        out_type=jax.ShapeDtypeStruct((NC, n_pad, d), jnp.float32),
        mesh=mesh,
        scratch_types=scratch,
    )


def _make_sc_degree(n, e):
    per_w = e // NW
    k, nchunk = _chunking(per_w)
    n_pad = ((n + 2047) // 2048) * 2048
    rpt = n_pad // NS

    mesh = plsc.VectorSubcoreMesh(core_axis_name="c", subcore_axis_name="s")

    scratch = [
        pltpu.VMEM((nchunk, k), jnp.int32),
        pltpu.VMEM((k,), jnp.float32),
        pltpu.VMEM((rpt,), jnp.float32),
        pltpu.VMEM_SHARED((n_pad,), jnp.float32),
        pltpu.SemaphoreType.DMA,
    ]

    def body(dst_hbm, out_hbm, dst_v, ones_v, zbuf, acc, sem):
        c = lax.axis_index("c")
        s = lax.axis_index("s")
        wid = c * NS + s

        pltpu.sync_copy(dst_hbm.at[wid], dst_v)

        def fill1(r, carry):
            ones_v[pl.ds(r * LANES, LANES)] = jnp.ones((LANES,), jnp.float32)
            return carry
        lax.fori_loop(0, k // LANES, fill1, 0)

        def fill0(r, carry):
            zbuf[pl.ds(r * LANES, LANES)] = jnp.zeros((LANES,), jnp.float32)
            return carry
        lax.fori_loop(0, rpt // LANES, fill0, 0)

        r0 = s * rpt
        pltpu.sync_copy(zbuf, acc.at[pl.ds(r0, rpt)])
        plsc.subcore_barrier()

        def chunk(j, carry):
            pltpu.sync_copy(ones_v, acc.at[dst_v.at[j]], add=True)
            return carry

        lax.fori_loop(0, nchunk, chunk, 0)
        plsc.subcore_barrier()

        pltpu.sync_copy(acc.at[pl.ds(r0, rpt)], zbuf)
        pltpu.sync_copy(zbuf, out_hbm.at[c].at[pl.ds(r0, rpt)])

    return pl.kernel(
        body,
        out_type=jax.ShapeDtypeStruct((NC, n_pad), jnp.float32),
        mesh=mesh,
        scratch_types=scratch,
    )


def _rowscale(xblk, dis_t):
    g = dis_t.shape[1]
    parts = [xblk[128 * i:128 * (i + 1), :] * dis_t[:, i:i + 1] for i in range(g)]
    return jnp.concatenate(parts, axis=0)


def _tc_a0_body(x_ref, w1_ref, o_ref):
    o_ref[...] = jnp.dot(x_ref[...], w1_ref[...],
                         preferred_element_type=jnp.float32)


def _tc_a1_body(xw_ref, degp_ref, hs1_ref, dist_ref):
    deg = 1.0 + degp_ref[0] + degp_ref[1]
    dis_t = jnp.transpose(1.0 / jnp.sqrt(deg))
    dist_ref[0] = dis_t
    hs1_ref[...] = _rowscale(xw_ref[...], dis_t)


def _tc_b_body(a_ref, hs1_ref, dist_ref, b1_ref, wc_ref, o_ref, dh):
    dis_t = dist_ref[0]
    agg = a_ref[0] + a_ref[1] + hs1_ref[...]
    h = jnp.maximum(_rowscale(agg, dis_t) + b1_ref[...], 0.0)
    hw = jnp.dot(h, wc_ref[...], preferred_element_type=jnp.float32)
    hs2 = _rowscale(hw, dis_t)
    o_ref[...] = jnp.concatenate([hs2] * (dh // hs2.shape[1]), axis=1)


def _tc_c_body(a_ref, hs2_ref, dist_ref, bc_ref, mu_ref, lv_ref, dz):
    dis_t = dist_ref[0]
    agg = a_ref[0] + a_ref[1]
    mu_ref[...] = _rowscale(agg[:, :dz] + hs2_ref[:, :dz], dis_t) + bc_ref[:, :dz]
    lv_ref[...] = _rowscale(agg[:, dz:2 * dz] + hs2_ref[:, dz:2 * dz],
                            dis_t) + bc_ref[:, dz:]


def _tc_adj_body(zi_ref, zj_ref, o_ref):
    logits = lax.dot_general(
        zi_ref[...], zj_ref[...], (((1,), (1,)), ((), ())),
        preferred_element_type=jnp.float32)
    o_ref[...] = jax.nn.sigmoid(logits)


def kernel(x, edge_index, W1, b1, Wmu, bmu, Wlv, blv):
    n, din = x.shape
    e = edge_index.shape[1]
    dh = W1.shape[1]
    dz = Wmu.shape[1]
    d2 = 2 * dz

    k, nchunk = _chunking(e // NW)
    nh, snc = _slabbing(nchunk)
    src_w = edge_index[0].reshape(NW, nh, snc, k)
    dst_w = edge_index[1].reshape(NW, nh, snc, k)
    dst_flat = edge_index[1].reshape(NW, nchunk, k)

    wc = jnp.concatenate([Wmu, Wlv], axis=1)
    bc = jnp.concatenate([bmu, blv]).reshape(1, d2)
    b1r = b1.reshape(1, dh)

    n_pad_deg = ((n + 2047) // 2048) * 2048

    degp = _make_sc_degree(n, e)(dst_flat)
    degp = degp.reshape(NC, n_pad_deg // 128, 128)

    bm = 1024
    g = bm // 128
    gi = pl.cdiv(n, bm)
    row = lambda i: (i, 0)
    halfrow = lambda i: (0, i, 0)

    xw = pl.pallas_call(
        _tc_a0_body,
        grid=(gi,),
        in_specs=[
            pl.BlockSpec((bm, din), row),
            pl.BlockSpec((din, dh), lambda i: (0, 0)),
        ],
        out_specs=pl.BlockSpec((bm, dh), row),
        out_shape=jax.ShapeDtypeStruct((n, dh), jnp.float32),
    )(x, W1)

    hs1, dist = pl.pallas_call(
        _tc_a1_body,
        grid=(gi,),
        in_specs=[
            pl.BlockSpec((bm, dh), row),
            pl.BlockSpec((NC, g, 128), halfrow),
        ],
        out_specs=[
            pl.BlockSpec((bm, dh), row),
            pl.BlockSpec((1, 128, g), lambda i: (i, 0, 0)),
        ],
        out_shape=[
            jax.ShapeDtypeStruct((n, dh), jnp.float32),
            jax.ShapeDtypeStruct((gi, 128, g), jnp.float32),
        ],
    )(xw, degp)

    acc1 = _make_sc_scatter(n, e, dh)(hs1, src_w, dst_w)

    hs2 = pl.pallas_call(
        functools.partial(_tc_b_body, dh=dh),
        grid=(gi,),
        in_specs=[
            pl.BlockSpec((NC, bm, dh), halfrow),
            pl.BlockSpec((bm, dh), row),
            pl.BlockSpec((1, 128, g), lambda i: (i, 0, 0)),
            pl.BlockSpec((1, dh), lambda i: (0, 0)),
            pl.BlockSpec((dh, d2), lambda i: (0, 0)),
        ],
        out_specs=pl.BlockSpec((bm, dh), row),
        out_shape=jax.ShapeDtypeStruct((n, dh), jnp.float32),
    )(acc1, hs1, dist, b1r, wc)

    acc2 = _make_sc_scatter(n, e, dh)(hs2, src_w, dst_w)

    mu, logvar = pl.pallas_call(
        functools.partial(_tc_c_body, dz=dz),
        grid=(gi,),
        in_specs=[
            pl.BlockSpec((NC, bm, dh), halfrow),
            pl.BlockSpec((bm, dh), row),
            pl.BlockSpec((1, 128, g), lambda i: (i, 0, 0)),
            pl.BlockSpec((1, d2), lambda i: (0, 0)),
        ],
        out_specs=[
            pl.BlockSpec((bm, dz), row),
            pl.BlockSpec((bm, dz), row),
        ],
        out_shape=[
            jax.ShapeDtypeStruct((n, dz), jnp.float32),
            jax.ShapeDtypeStruct((n, dz), jnp.float32),
        ],
    )(acc2, hs2, dist, bc)

    bi, bn = 2048, 2560
    gia = pl.cdiv(n, bi)
    gj = pl.cdiv(n, bn)
    adj = pl.pallas_call(
        _tc_adj_body,
        grid=(gia, gj),
        in_specs=[
            pl.BlockSpec((bi, dz), lambda i, j: (i, 0)),
            pl.BlockSpec((bn, dz), lambda i, j: (j, 0)),
        ],
        out_specs=pl.BlockSpec((bi, bn), lambda i, j: (i, j)),
        out_shape=jax.ShapeDtypeStruct((n, n), jnp.float32),
    )(mu, mu)

    return (mu, logvar, mu, adj)


# speedup vs baseline: 21.5578x; 1.0105x over previous
import functools

import jax
import jax.numpy as jnp
from jax import lax
from jax.experimental import pallas as pl
from jax.experimental.pallas import tpu as pltpu
from jax.experimental.pallas import tpu_sc as plsc

NC, NS, LANES = 2, 16, 16
NW = NC * NS


def _fill_rows(ref, rows, d, val):
    vec = jnp.full((LANES,), val, jnp.float32)

    def body(r, carry):
        for j in range(d // LANES):
            ref[r, pl.ds(j * LANES, LANES)] = vec
        return carry

    lax.fori_loop(0, rows, body, 0)


def _chunking(per_w):
    k = 80 if per_w % 80 == 0 else 8
    return k, per_w // k


def _slabbing(nchunk):
    h = 5 if nchunk % 5 == 0 else 1
    return h, nchunk // h


def _make_sc_scatter(n, e, d):
    per_w = e // NW
    k, nchunk = _chunking(per_w)
    nh, snc = _slabbing(nchunk)
    n_pad = ((n + 127) // 128) * 128
    rpt = n_pad // NS

    mesh = plsc.VectorSubcoreMesh(core_axis_name="c", subcore_axis_name="s")

    zr = 8

    nb = 4

    scratch = [
        pltpu.VMEM((snc, k), jnp.int32),
        pltpu.VMEM((snc, k), jnp.int32),
    ] + [pltpu.VMEM((k, d), jnp.float32)] * nb + [
        pltpu.VMEM((zr, d), jnp.float32),
        pltpu.VMEM_SHARED((n_pad, d), jnp.float32),
    ] + [pltpu.SemaphoreType.DMA] * nb

    def body(vals_hbm, src_hbm, dst_hbm, out_hbm, src_v, dst_v, *rest):
        rows = rest[:nb]
        zbuf, acc = rest[nb], rest[nb + 1]
        sems = rest[nb + 2:]
        c = lax.axis_index("c")
        s = lax.axis_index("s")
        wid = c * NS + s

        _fill_rows(zbuf, zr, d, 0.0)
        r0 = s * rpt
        nz = rpt // zr
        for b in range(0, nz, nb):
            hz = [pltpu.async_copy(zbuf, acc.at[pl.ds(r0 + (b + t) * zr, zr)],
                                   sems[t])
                  for t in range(min(nb, nz - b))]
            for h in hz:
                h.wait()
        plsc.subcore_barrier()

        def group(i, carry):
            j = nb * i
            hs = [pltpu.async_copy(vals_hbm.at[src_v.at[j + t]], rows[t], sems[t])
                  for t in range(nb)]
            for t in range(nb):
                hs[t].wait()
                pltpu.sync_copy(rows[t], acc.at[dst_v.at[j + t]], add=True)
            return carry

        def slab(h, carry):
            pltpu.sync_copy(src_hbm.at[wid].at[h], src_v)
            pltpu.sync_copy(dst_hbm.at[wid].at[h], dst_v)
            lax.fori_loop(0, snc // nb, group, 0)
            for j in range(nb * (snc // nb), snc):
                t = j % nb
                pltpu.async_copy(vals_hbm.at[src_v.at[j]], rows[t], sems[t]).wait()
                pltpu.sync_copy(rows[t], acc.at[dst_v.at[j]], add=True)
            return carry

        lax.fori_loop(0, nh, slab, 0)
        plsc.subcore_barrier()

        pltpu.sync_copy(acc.at[pl.ds(r0, rpt)], out_hbm.at[c].at[pl.ds(r0, rpt)])

    return pl.kernel(
        body,
        out_type=jax.ShapeDtypeStruct((NC, n_pad, d), jnp.float32),
        mesh=mesh,
        scratch_types=scratch,
    )


def _make_sc_degree(n, e):
    per_w = e // NW
    k, nchunk = _chunking(per_w)
    n_pad = ((n + 2047) // 2048) * 2048
    rpt = n_pad // NS

    mesh = plsc.VectorSubcoreMesh(core_axis_name="c", subcore_axis_name="s")

    scratch = [
        pltpu.VMEM((nchunk, k), jnp.int32),
        pltpu.VMEM((k,), jnp.float32),
        pltpu.VMEM((rpt,), jnp.float32),
        pltpu.VMEM_SHARED((n_pad,), jnp.float32),
        pltpu.SemaphoreType.DMA,
    ]

    def body(dst_hbm, out_hbm, dst_v, ones_v, zbuf, acc, sem):
        c = lax.axis_index("c")
        s = lax.axis_index("s")
        wid = c * NS + s

        pltpu.sync_copy(dst_hbm.at[wid], dst_v)

        def fill1(r, carry):
            ones_v[pl.ds(r * LANES, LANES)] = jnp.ones((LANES,), jnp.float32)
            return carry
        lax.fori_loop(0, k // LANES, fill1, 0)

        def fill0(r, carry):
            zbuf[pl.ds(r * LANES, LANES)] = jnp.zeros((LANES,), jnp.float32)
            return carry
        lax.fori_loop(0, rpt // LANES, fill0, 0)

        r0 = s * rpt
        pltpu.sync_copy(zbuf, acc.at[pl.ds(r0, rpt)])
        plsc.subcore_barrier()

        def chunk(j, carry):
            pltpu.sync_copy(ones_v, acc.at[dst_v.at[j]], add=True)
            return carry

        lax.fori_loop(0, nchunk, chunk, 0)
        plsc.subcore_barrier()

        pltpu.sync_copy(acc.at[pl.ds(r0, rpt)], zbuf)
        pltpu.sync_copy(zbuf, out_hbm.at[c].at[pl.ds(r0, rpt)])

    return pl.kernel(
        body,
        out_type=jax.ShapeDtypeStruct((NC, n_pad), jnp.float32),
        mesh=mesh,
        scratch_types=scratch,
    )


def _rowscale(xblk, dis_t):
    g = dis_t.shape[1]
    parts = [xblk[128 * i:128 * (i + 1), :] * dis_t[:, i:i + 1] for i in range(g)]
    return jnp.concatenate(parts, axis=0)


def _tc_a_body(x_ref, w1_ref, degp_ref, hs1_ref, dist_ref):
    deg = 1.0 + degp_ref[0] + degp_ref[1]
    dis_t = jnp.transpose(1.0 / jnp.sqrt(deg))
    dist_ref[0] = dis_t
    xw = jnp.dot(x_ref[...], w1_ref[...], preferred_element_type=jnp.float32)
    hs1_ref[...] = _rowscale(xw, dis_t)


def _tc_b_body(a_ref, hs1_ref, dist_ref, b1_ref, wc_ref, o_ref, dh):
    dis_t = dist_ref[0]
    agg = a_ref[0] + a_ref[1] + hs1_ref[...]
    h = jnp.maximum(_rowscale(agg, dis_t) + b1_ref[...], 0.0)
    hw = jnp.dot(h, wc_ref[...], preferred_element_type=jnp.float32)
    hs2 = _rowscale(hw, dis_t)
    o_ref[...] = jnp.concatenate([hs2] * (dh // hs2.shape[1]), axis=1)


def _tc_c_body(a_ref, hs2_ref, dist_ref, bc_ref, mu_ref, lv_ref, dz):
    dis_t = dist_ref[0]
    agg = a_ref[0] + a_ref[1]
    mu_ref[...] = _rowscale(agg[:, :dz] + hs2_ref[:, :dz], dis_t) + bc_ref[:, :dz]
    lv_ref[...] = _rowscale(agg[:, dz:2 * dz] + hs2_ref[:, dz:2 * dz],
                            dis_t) + bc_ref[:, dz:]


def _tc_adj_body(zi_ref, zj_ref, o_ref):
    logits = lax.dot_general(
        zi_ref[...], zj_ref[...], (((1,), (1,)), ((), ())),
        preferred_element_type=jnp.float32)
    o_ref[...] = jax.nn.sigmoid(logits)


def kernel(x, edge_index, W1, b1, Wmu, bmu, Wlv, blv):
    n, din = x.shape
    e = edge_index.shape[1]
    dh = W1.shape[1]
    dz = Wmu.shape[1]
    d2 = 2 * dz

    k, nchunk = _chunking(e // NW)
    nh, snc = _slabbing(nchunk)
    src_w = edge_index[0].reshape(NW, nh, snc, k)
    dst_w = edge_index[1].reshape(NW, nh, snc, k)
    dst_flat = edge_index[1].reshape(NW, nchunk, k)

    wc = jnp.concatenate([Wmu, Wlv], axis=1)
    bc = jnp.concatenate([bmu, blv]).reshape(1, d2)
    b1r = b1.reshape(1, dh)

    n_pad_deg = ((n + 2047) // 2048) * 2048

    degp = _make_sc_degree(n, e)(dst_flat)
    degp = degp.reshape(NC, n_pad_deg // 128, 128)

    bm = 1024
    g = bm // 128
    gi = pl.cdiv(n, bm)
    row = lambda i: (i, 0)
    halfrow = lambda i: (0, i, 0)

    hs1, dist = pl.pallas_call(
        _tc_a_body,
        grid=(gi,),
        in_specs=[
            pl.BlockSpec((bm, din), row),
            pl.BlockSpec((din, dh), lambda i: (0, 0)),
            pl.BlockSpec((NC, g, 128), halfrow),
        ],
        out_specs=[
            pl.BlockSpec((bm, dh), row),
            pl.BlockSpec((1, 128, g), lambda i: (i, 0, 0)),
        ],
        out_shape=[
            jax.ShapeDtypeStruct((n, dh), jnp.float32),
            jax.ShapeDtypeStruct((gi, 128, g), jnp.float32),
        ],
    )(x, W1, degp)

    acc1 = _make_sc_scatter(n, e, dh)(hs1, src_w, dst_w)

    hs2 = pl.pallas_call(
        functools.partial(_tc_b_body, dh=dh),
        grid=(gi,),
        in_specs=[
            pl.BlockSpec((NC, bm, dh), halfrow),
            pl.BlockSpec((bm, dh), row),
            pl.BlockSpec((1, 128, g), lambda i: (i, 0, 0)),
            pl.BlockSpec((1, dh), lambda i: (0, 0)),
            pl.BlockSpec((dh, d2), lambda i: (0, 0)),
        ],
        out_specs=pl.BlockSpec((bm, dh), row),
        out_shape=jax.ShapeDtypeStruct((n, dh), jnp.float32),
    )(acc1, hs1, dist, b1r, wc)

    acc2 = _make_sc_scatter(n, e, dh)(hs2, src_w, dst_w)

    mu, logvar = pl.pallas_call(
        functools.partial(_tc_c_body, dz=dz),
        grid=(gi,),
        in_specs=[
            pl.BlockSpec((NC, bm, dh), halfrow),
            pl.BlockSpec((bm, dh), row),
            pl.BlockSpec((1, 128, g), lambda i: (i, 0, 0)),
            pl.BlockSpec((1, d2), lambda i: (0, 0)),
        ],
        out_specs=[
            pl.BlockSpec((bm, dz), row),
            pl.BlockSpec((bm, dz), row),
        ],
        out_shape=[
            jax.ShapeDtypeStruct((n, dz), jnp.float32),
            jax.ShapeDtypeStruct((n, dz), jnp.float32),
        ],
    )(acc2, hs2, dist, bc)

    bi, bn = 2048, 2560
    gia = pl.cdiv(n, bi)
    gj = pl.cdiv(n, bn)
    adj = pl.pallas_call(
        _tc_adj_body,
        grid=(gia, gj),
        in_specs=[
            pl.BlockSpec((bi, dz), lambda i, j: (i, 0)),
            pl.BlockSpec((bn, dz), lambda i, j: (j, 0)),
        ],
        out_specs=pl.BlockSpec((bi, bn), lambda i, j: (i, j)),
        out_shape=jax.ShapeDtypeStruct((n, n), jnp.float32),
    )(mu, mu)

    return (mu, logvar, mu, adj)
